# baseline (speedup 1.0000x reference)
import functools

import jax
import jax.numpy as jnp
from jax import lax
from jax.experimental import pallas as pl
from jax.experimental.pallas import tpu as pltpu
from jax.experimental.pallas import tpu_sc as plsc

B = 8
NPG = 1250
NP = 1280
N = B * NPG
A = 5
T = 10
EPG = 20000
E = B * EPG
IN = 4
HC = 16
H = 8
D = H * HC
HID = HC * 2
OUT = 2

NC = 2
NS = 16
NW = NC * NS

_SC_MESH = dict(core_axis_name="c", subcore_axis_name="s")


def _wid():
    return lax.axis_index("s") * NC + lax.axis_index("c")


def _zero_vec(ref, n16):
    z = jnp.zeros((16,), jnp.float32)

    @plsc.parallel_loop(0, n16, 1, unroll=8)
    def zb(i):
        ref[pl.ds(i * 16, 16)] = z


def _make_loopattr():
    mesh = plsc.VectorSubcoreMesh(**_SC_MESH)
    units = T * B

    @functools.partial(
        pl.kernel,
        out_type=jax.ShapeDtypeStruct((T, B, NP), jnp.float32),
        mesh=mesh,
        scratch_types=[
            pltpu.VMEM((EPG,), jnp.int32),
            pltpu.VMEM((EPG,), jnp.float32),
            pltpu.VMEM((NP,), jnp.float32),
            pltpu.VMEM((NP,), jnp.float32),
        ],
        compiler_params=pltpu.CompilerParams(needs_layout_passes=False, use_tc_tiling_on_sc=False),
    )
    def loopattr(dst_h, ea_h, la_h, dst_v, ea_v, deg_v, eas_v):
        wid = _wid()
        ones = jnp.ones((16,), jnp.float32)

        def unit(k, _):
            uid = wid + NW * k

            @pl.when(uid < units)
            def _():
                t = uid // B
                g = uid - t * B
                pltpu.sync_copy(dst_h.at[t, g], dst_v)
                pltpu.sync_copy(ea_h.at[t, g], ea_v)
                _zero_vec(deg_v, NP // 16)
                _zero_vec(eas_v, NP // 16)

                @plsc.parallel_loop(0, EPG // 16, 1, unroll=4)
                def eb(i):
                    sl = pl.ds(i * 16, 16)
                    dv = dst_v[sl]
                    plsc.addupdate_scatter(deg_v, [dv], ones)
                    plsc.addupdate_scatter(eas_v, [dv], ea_v[sl])

                @plsc.parallel_loop(0, NP // 16, 1, unroll=2)
                def nb(i):
                    sl = pl.ds(i * 16, 16)
                    eas_v[sl] = eas_v[sl] / jnp.maximum(deg_v[sl], 1.0)

                pltpu.sync_copy(eas_v, la_h.at[t, g])

            return 0

        lax.fori_loop(0, (units + NW - 1) // NW, unit, 0)

    return loopattr


def _make_gat(self_loops: bool):
    mesh = plsc.VectorSubcoreMesh(**_SC_MESH)
    scratch = [
        pltpu.VMEM((EPG,), jnp.int32),
        pltpu.VMEM((EPG,), jnp.int32),
        pltpu.VMEM((EPG,), jnp.float32),
        pltpu.VMEM((EPG,), jnp.float32),
        pltpu.VMEM((HC, NP), jnp.float32),
        pltpu.VMEM((HC, NP), jnp.float32),
        pltpu.VMEM((NP,), jnp.float32),
        pltpu.VMEM((NP,), jnp.float32),
        pltpu.VMEM((NP,), jnp.float32),
        pltpu.VMEM((HC,), jnp.float32),
    ]
    if self_loops:
        scratch += [
            pltpu.VMEM((NP,), jnp.float32),
            pltpu.VMEM((NP,), jnp.float32),
        ]

    def body(src_h, dst_h, ea_h, asrc_h, adst_h, xs_h, kap_h, *rest):
        if self_loops:
            (la_h, out_h, src_v, dst_v, ea_v, ex_v, xs_v, out_v, asrc_v,
             adst_v, den_v, kap_v, la_v, exl_v) = rest
        else:
            (out_h, src_v, dst_v, ea_v, ex_v, xs_v, out_v, asrc_v,
             adst_v, den_v, kap_v) = rest
        wid = _wid()

        def unit(k, _):
            uid = wid + NW * k
            t = uid // (H * B)
            r = uid - t * (H * B)
            h = r // B
            g = r - h * B

            pltpu.sync_copy(src_h.at[t, g], src_v)
            pltpu.sync_copy(dst_h.at[t, g], dst_v)
            pltpu.sync_copy(ea_h.at[t, g], ea_v)
            pltpu.sync_copy(asrc_h.at[t, h, g], asrc_v)
            pltpu.sync_copy(adst_h.at[t, h, g], adst_v)
            pltpu.sync_copy(xs_h.at[t, h, g], xs_v)
            pltpu.sync_copy(kap_h.at[h], kap_v)
            if self_loops:
                pltpu.sync_copy(la_h.at[t, g], la_v)
            kap = kap_v[...]

            _zero_vec(den_v, NP // 16)
            z = jnp.zeros((16,), jnp.float32)

            @plsc.parallel_loop(0, NP // 16, 1, unroll=2)
            def zo(i):
                sl = pl.ds(i * 16, 16)
                for c in range(HC):
                    out_v[c, sl] = z

            @plsc.parallel_loop(0, EPG // 16, 1, unroll=4,
                                carry=jnp.full((16,), -1e30, jnp.float32))
            def pa(i, m):
                sl = pl.ds(i * 16, 16)
                a = (plsc.load_gather(asrc_v, [src_v[sl]])
                     + plsc.load_gather(adst_v, [dst_v[sl]])
                     + ea_v[sl] * kap)
                alpha = jnp.maximum(a, 0.2 * a)
                ex_v[sl] = alpha
                return jnp.maximum(m, alpha)

            m = pa
            if self_loops:
                @plsc.parallel_loop(0, NP // 16, 1, unroll=2, carry=m)
                def pal(i, mm):
                    sl = pl.ds(i * 16, 16)
                    a = asrc_v[sl] + adst_v[sl] + la_v[sl] * kap
                    alpha = jnp.maximum(a, 0.2 * a)
                    exl_v[sl] = alpha
                    return jnp.maximum(mm, alpha)

                m = pal
            gmax = jnp.full((16,), jnp.max(m), jnp.float32)

            @plsc.parallel_loop(0, EPG // 16, 1, unroll=4)
            def pbc(i):
                sl = pl.ds(i * 16, 16)
                ev = jnp.exp(ex_v[sl] - gmax)
                sv = src_v[sl]
                dv = dst_v[sl]
                plsc.addupdate_scatter(den_v, [dv], ev)
                for c in range(HC):
                    cv = jnp.full((16,), c, jnp.int32)
                    gv = plsc.load_gather(xs_v, [cv, sv])
                    plsc.addupdate_scatter(out_v, [cv, dv], gv * ev)

            if self_loops:
                @plsc.parallel_loop(0, NP // 16, 1, unroll=2)
                def pbcl(i):
                    sl = pl.ds(i * 16, 16)
                    ev = jnp.exp(exl_v[sl] - gmax)
                    den_v[sl] = den_v[sl] + ev
                    for c in range(HC):
                        out_v[c, sl] = out_v[c, sl] + xs_v[c, sl] * ev

            @plsc.parallel_loop(0, NP // 16, 1, unroll=2)
            def pd(i):
                sl = pl.ds(i * 16, 16)
                rv = 1.0 / (den_v[sl] + 1e-16)
                for c in range(HC):
                    out_v[c, sl] = out_v[c, sl] * rv

            pltpu.sync_copy(out_v, out_h.at[t, h, g])
            return 0

        lax.fori_loop(0, (T * H * B) // NW, unit, 0)

    return functools.partial(
        pl.kernel,
        out_type=jax.ShapeDtypeStruct((T, H, B, HC, NP), jnp.float32),
        mesh=mesh,
        scratch_types=scratch,
        compiler_params=pltpu.CompilerParams(needs_layout_passes=False, use_tc_tiling_on_sc=False),
    )(body)


_BLK = 2000


def _dense_tc(xin, w, As, Ad, b=None):
    din = xin.shape[-1]
    has_b = b is not None

    def body(*refs):
        if has_b:
            x_ref, w_ref, as_ref, ad_ref, b_ref, xs_ref, s_ref, d_ref = refs
        else:
            x_ref, w_ref, as_ref, ad_ref, xs_ref, s_ref, d_ref = refs
        xv = x_ref[0]
        if has_b:
            xv = jnp.maximum(xv + b_ref[...], 0.0)
        xs = jnp.dot(xv, w_ref[...], preferred_element_type=jnp.float32)
        xs_ref[0] = xs
        s_ref[0] = jnp.dot(xs, as_ref[...], preferred_element_type=jnp.float32)
        d_ref[0] = jnp.dot(xs, ad_ref[...], preferred_element_type=jnp.float32)

    in_specs = [
        pl.BlockSpec((1, _BLK, din), lambda t, i: (t, i, 0)),
        pl.BlockSpec((din, D), lambda t, i: (0, 0)),
        pl.BlockSpec((D, H), lambda t, i: (0, 0)),
        pl.BlockSpec((D, H), lambda t, i: (0, 0)),
    ]
    args = [xin, w, As, Ad]
    if has_b:
        in_specs.append(pl.BlockSpec((din,), lambda t, i: (0,)))
        args.append(b)
    return pl.pallas_call(
        body,
        grid=(T, N // _BLK),
        in_specs=in_specs,
        out_specs=[
            pl.BlockSpec((1, _BLK, D), lambda t, i: (t, i, 0)),
            pl.BlockSpec((1, _BLK, H), lambda t, i: (t, i, 0)),
            pl.BlockSpec((1, _BLK, H), lambda t, i: (t, i, 0)),
        ],
        out_shape=[
            jax.ShapeDtypeStruct((T, N, D), jnp.float32),
            jax.ShapeDtypeStruct((T, N, H), jnp.float32),
            jax.ShapeDtypeStruct((T, N, H), jnp.float32),
        ],
    )(*args)


def _pool_gru_fc(out3, b3, wih_t, whh_t, bih, bhh, f1w_t, f1b, f2w_t, f2b):
    def body(o_ref, b3_ref, wih_ref, whh_ref, bih_ref, bhh_ref, f1w_ref,
             f1b_ref, f2w_ref, f2b_ref, pred_ref, h_ref):
        t = pl.program_id(0)

        @pl.when(t == 0)
        def _():
            h_ref[...] = jnp.zeros_like(h_ref)

        xt = jnp.maximum(o_ref[0] + b3_ref[...], 0.0)
        x3 = xt.reshape(B, NPG, D)
        gmean = jnp.sum(x3, axis=1) * (1.0 / NPG)
        ag = x3[:, :A, :]
        gr = jnp.broadcast_to(gmean[:, None, :], (B, A, D))
        comb = jnp.concatenate([ag, gr], axis=-1).reshape(B * A, 2 * D)
        hprev = h_ref[...]
        gi = jnp.dot(comb, wih_ref[...],
                     preferred_element_type=jnp.float32) + bih_ref[...]
        gh = jnp.dot(hprev, whh_ref[...],
                     preferred_element_type=jnp.float32) + bhh_ref[...]
        ir, iz, inn = gi[:, :HID], gi[:, HID:2 * HID], gi[:, 2 * HID:]
        hr, hz, hn = gh[:, :HID], gh[:, HID:2 * HID], gh[:, 2 * HID:]
        rg = jax.nn.sigmoid(ir + hr)
        zg = jax.nn.sigmoid(iz + hz)
        ng = jnp.tanh(inn + rg * hn)
        hnew = (1.0 - zg) * ng + zg * hprev
        h_ref[...] = hnew
        f1 = jnp.maximum(
            jnp.dot(hnew, f1w_ref[...],
                    preferred_element_type=jnp.float32) + f1b_ref[...], 0.0)
        pred = jnp.dot(f1, f2w_ref[...],
                       preferred_element_type=jnp.float32) + f2b_ref[...]
        pred_ref[0] = pred.reshape(B, A, OUT)

    full = lambda shape: pl.BlockSpec(shape, lambda t: tuple(0 for _ in shape))
    return pl.pallas_call(
        body,
        grid=(T,),
        in_specs=[
            pl.BlockSpec((1, N, D), lambda t: (t, 0, 0)),
            full((D,)),
            full((2 * D, 3 * HID)),
            full((HID, 3 * HID)),
            full((3 * HID,)),
            full((3 * HID,)),
            full((HID, HC)),
            full((HC,)),
            full((HC, OUT)),
            full((OUT,)),
        ],
        out_specs=pl.BlockSpec((1, B, A, OUT), lambda t: (t, 0, 0, 0)),
        out_shape=jax.ShapeDtypeStruct((T, B, A, OUT), jnp.float32),
        scratch_shapes=[pltpu.VMEM((B * A, HID), jnp.float32)],
    )(out3, b3, wih_t, whh_t, bih, bhh, f1w_t, f1b, f2w_t, f2b)


def _to_sc_nodes(a):
    a = a.reshape(T, B, NPG, H).transpose(0, 3, 1, 2)
    return jnp.pad(a, ((0, 0), (0, 0), (0, 0), (0, NP - NPG)))


def _to_sc_feats(xs):
    v = xs.reshape(T, B, NPG, H, HC).transpose(0, 3, 1, 4, 2)
    return jnp.pad(v, ((0, 0), (0, 0), (0, 0), (0, 0), (0, NP - NPG)))


def _from_sc(o):
    o = o[..., :NPG]
    return o.transpose(0, 2, 4, 1, 3).reshape(T, N, D)


def _block_diag_proj(a):
    return (jnp.eye(H, dtype=a.dtype)[:, None, :] * a[:, :, None]).reshape(D, H)


_loopattr_k = _make_loopattr()
_gat_nl = _make_gat(False)
_gat_sl = _make_gat(True)


def kernel(x, edge_index, edge_attr, batch, w1, as1, ad1, ae1, we1, b1, w2,
           as2, ad2, ae2, we2, b2, w3, as3, ad3, ae3, we3, b3, wih, whh, bih,
           bhh, fc1w, fc1b, fc2w, fc2b):
    offs = (jnp.arange(B, dtype=jnp.int32) * NPG)[None, :, None]
    src_h = edge_index[:, 0].reshape(T, B, EPG) - offs
    dst_h = edge_index[:, 1].reshape(T, B, EPG) - offs
    ea_h = edge_attr.reshape(T, B, EPG)

    kaps = []
    for we_l, ae_l in ((we1, ae1), (we2, ae2), (we3, ae3)):
        kap = (we_l.reshape(H, HC) * ae_l).sum(-1)
        kaps.append(jnp.broadcast_to(kap[:, None], (H, HC)))

    la_h = _loopattr_k(dst_h, ea_h)

    xin = x
    layers = (
        (w1, as1, ad1, None, kaps[0], False),
        (w2, as2, ad2, b1, kaps[1], True),
        (w3, as3, ad3, b2, kaps[2], True),
    )
    for w_l, as_l, ad_l, b_prev, kap_l, sl in layers:
        xs, asrc, adst = _dense_tc(xin, w_l, _block_diag_proj(as_l),
                                   _block_diag_proj(ad_l), b_prev)
        a_s = _to_sc_nodes(asrc)
        a_d = _to_sc_nodes(adst)
        xs_sc = _to_sc_feats(xs)
        if sl:
            o = _gat_sl(src_h, dst_h, ea_h, a_s, a_d, xs_sc, kap_l, la_h)
        else:
            o = _gat_nl(src_h, dst_h, ea_h, a_s, a_d, xs_sc, kap_l)
        xin = _from_sc(o)

    preds = _pool_gru_fc(xin, b3, wih.T, whh.T, bih, bhh, fc1w.T, fc1b,
                         fc2w.T, fc2b)
    return preds.transpose(1, 0, 2, 3)


# speedup vs baseline: 1.8413x; 1.8413x over previous
import functools

import jax
import jax.numpy as jnp
from jax import lax
from jax.experimental import pallas as pl
from jax.experimental.pallas import tpu as pltpu
from jax.experimental.pallas import tpu_sc as plsc

B = 8
NPG = 1250
NP = 1280
N = B * NPG
A = 5
T = 10
EPG = 20000
E = B * EPG
IN = 4
HC = 16
H = 8
D = H * HC
HID = HC * 2
OUT = 2

NC = 2
NS = 16
NW = NC * NS

_SC_MESH = dict(core_axis_name="c", subcore_axis_name="s")


def _wid():
    return lax.axis_index("s") * NC + lax.axis_index("c")


def _zero_vec(ref, n16):
    z = jnp.zeros((16,), jnp.float32)

    @plsc.parallel_loop(0, n16, 1, unroll=8)
    def zb(i):
        ref[pl.ds(i * 16, 16)] = z


def _make_loopattr():
    mesh = plsc.VectorSubcoreMesh(**_SC_MESH)
    units = T * B

    @functools.partial(
        pl.kernel,
        out_type=jax.ShapeDtypeStruct((T, B, NP), jnp.float32),
        mesh=mesh,
        scratch_types=[
            pltpu.VMEM((EPG,), jnp.int32),
            pltpu.VMEM((EPG,), jnp.float32),
            pltpu.VMEM((NP,), jnp.float32),
            pltpu.VMEM((NP,), jnp.float32),
        ],
        compiler_params=pltpu.CompilerParams(needs_layout_passes=False, use_tc_tiling_on_sc=False),
    )
    def loopattr(dst_h, ea_h, la_h, dst_v, ea_v, deg_v, eas_v):
        wid = _wid()
        ones = jnp.ones((16,), jnp.float32)

        def unit(k, _):
            uid = wid + NW * k

            @pl.when(uid < units)
            def _():
                t = uid // B
                g = uid - t * B
                pltpu.sync_copy(dst_h.at[t, g], dst_v)
                pltpu.sync_copy(ea_h.at[t, g], ea_v)
                _zero_vec(deg_v, NP // 16)
                _zero_vec(eas_v, NP // 16)

                @plsc.parallel_loop(0, EPG // 16, 1, unroll=2)
                def eb(i):
                    sl = pl.ds(i * 16, 16)
                    dv = dst_v[sl]
                    plsc.addupdate_scatter(deg_v, [dv], ones)
                    plsc.addupdate_scatter(eas_v, [dv], ea_v[sl])

                @plsc.parallel_loop(0, NP // 16, 1, unroll=2)
                def nb(i):
                    sl = pl.ds(i * 16, 16)
                    eas_v[sl] = eas_v[sl] / jnp.maximum(deg_v[sl], 1.0)

                pltpu.sync_copy(eas_v, la_h.at[t, g])

            return 0

        lax.fori_loop(0, (units + NW - 1) // NW, unit, 0)

    return loopattr


def _make_gat(self_loops: bool):
    mesh = plsc.VectorSubcoreMesh(**_SC_MESH)
    scratch = [
        pltpu.VMEM((EPG,), jnp.int32),
        pltpu.VMEM((EPG,), jnp.int32),
        pltpu.VMEM((EPG,), jnp.float32),
        pltpu.VMEM((EPG,), jnp.float32),
        pltpu.VMEM((HC, NP), jnp.float32),
        pltpu.VMEM((HC, NP), jnp.float32),
        pltpu.VMEM((NP,), jnp.float32),
        pltpu.VMEM((NP,), jnp.float32),
        pltpu.VMEM((NP,), jnp.float32),
        pltpu.VMEM((HC,), jnp.float32),
    ]
    if self_loops:
        scratch += [
            pltpu.VMEM((NP,), jnp.float32),
            pltpu.VMEM((NP,), jnp.float32),
        ]

    def body(src_h, dst_h, ea_h, asrc_h, adst_h, xs_h, kap_h, *rest):
        if self_loops:
            (la_h, out_h, src_v, dst_v, ea_v, ex_v, xs_v, out_v, asrc_v,
             adst_v, den_v, kap_v, la_v, exl_v) = rest
        else:
            (out_h, src_v, dst_v, ea_v, ex_v, xs_v, out_v, asrc_v,
             adst_v, den_v, kap_v) = rest
        wid = _wid()

        def unit(k, _):
            uid = wid + NW * k
            t = uid // (H * B)
            r = uid - t * (H * B)
            h = r // B
            g = r - h * B

            pltpu.sync_copy(src_h.at[t, g], src_v)
            pltpu.sync_copy(dst_h.at[t, g], dst_v)
            pltpu.sync_copy(ea_h.at[t, g], ea_v)
            pltpu.sync_copy(asrc_h.at[t, g, h], asrc_v)
            pltpu.sync_copy(adst_h.at[t, g, h], adst_v)
            pltpu.sync_copy(xs_h.at[t, g, h], xs_v)
            pltpu.sync_copy(kap_h.at[h], kap_v)
            if self_loops:
                pltpu.sync_copy(la_h.at[t, g], la_v)
            kap = kap_v[...]

            _zero_vec(den_v, NP // 16)
            z = jnp.zeros((16,), jnp.float32)

            @plsc.parallel_loop(0, NP // 16, 1, unroll=2)
            def zo(i):
                sl = pl.ds(i * 16, 16)
                for c in range(HC):
                    out_v[c, sl] = z

            @plsc.parallel_loop(0, EPG // 16, 1, unroll=2,
                                carry=jnp.full((16,), -1e30, jnp.float32))
            def pa(i, m):
                sl = pl.ds(i * 16, 16)
                a = (plsc.load_gather(asrc_v, [src_v[sl]])
                     + plsc.load_gather(adst_v, [dst_v[sl]])
                     + ea_v[sl] * kap)
                alpha = jnp.maximum(a, 0.2 * a)
                ex_v[sl] = alpha
                return jnp.maximum(m, alpha)

            m = pa
            if self_loops:
                @plsc.parallel_loop(0, NP // 16, 1, unroll=2, carry=m)
                def pal(i, mm):
                    sl = pl.ds(i * 16, 16)
                    a = asrc_v[sl] + adst_v[sl] + la_v[sl] * kap
                    alpha = jnp.maximum(a, 0.2 * a)
                    exl_v[sl] = alpha
                    return jnp.maximum(mm, alpha)

                m = pal
            gmax = jnp.full((16,), jnp.max(m), jnp.float32)

            @plsc.parallel_loop(0, EPG // 16, 1, unroll=2)
            def pbc(i):
                sl = pl.ds(i * 16, 16)
                ev = jnp.exp(ex_v[sl] - gmax)
                sv = src_v[sl]
                dv = dst_v[sl]
                plsc.addupdate_scatter(den_v, [dv], ev)
                for c in range(HC):
                    cv = jnp.full((16,), c, jnp.int32)
                    gv = plsc.load_gather(xs_v, [cv, sv])
                    plsc.addupdate_scatter(out_v, [cv, dv], gv * ev)

            if self_loops:
                @plsc.parallel_loop(0, NP // 16, 1, unroll=2)
                def pbcl(i):
                    sl = pl.ds(i * 16, 16)
                    ev = jnp.exp(exl_v[sl] - gmax)
                    den_v[sl] = den_v[sl] + ev
                    for c in range(HC):
                        out_v[c, sl] = out_v[c, sl] + xs_v[c, sl] * ev

            @plsc.parallel_loop(0, NP // 16, 1, unroll=2)
            def pd(i):
                sl = pl.ds(i * 16, 16)
                rv = 1.0 / (den_v[sl] + 1e-16)
                for c in range(HC):
                    out_v[c, sl] = out_v[c, sl] * rv

            pltpu.sync_copy(out_v, out_h.at[t, g, h])
            return 0

        lax.fori_loop(0, (T * H * B) // NW, unit, 0)

    return functools.partial(
        pl.kernel,
        out_type=jax.ShapeDtypeStruct((T, B, H, HC, NP), jnp.float32),
        mesh=mesh,
        scratch_types=scratch,
        compiler_params=pltpu.CompilerParams(needs_layout_passes=False, use_tc_tiling_on_sc=False),
    )(body)


def _dense_tc(xin, wT, AsT, AdT, b=None):
    has_b = b is not None
    five_d = xin.ndim == 5
    din = wT.shape[1]

    def body(*refs):
        if has_b:
            x_ref, w_ref, as_ref, ad_ref, b_ref, xs_ref, s_ref, d_ref = refs
        else:
            x_ref, w_ref, as_ref, ad_ref, xs_ref, s_ref, d_ref = refs
        if five_d:
            xv = x_ref[0, 0].reshape(din, NP)
        else:
            xv = x_ref[0, 0]
        if has_b:
            xv = jnp.maximum(xv + b_ref[...].reshape(din, 1), 0.0)
        xs = jnp.dot(w_ref[...], xv, preferred_element_type=jnp.float32)
        xs_ref[0, 0] = xs.reshape(H, HC, NP)
        s_ref[0, 0] = jnp.dot(as_ref[...], xs,
                              preferred_element_type=jnp.float32)
        d_ref[0, 0] = jnp.dot(ad_ref[...], xs,
                              preferred_element_type=jnp.float32)

    if five_d:
        x_spec = pl.BlockSpec((1, 1, H, HC, NP), lambda t, g: (t, g, 0, 0, 0))
    else:
        x_spec = pl.BlockSpec((1, 1, din, NP), lambda t, g: (t, g, 0, 0))
    in_specs = [
        x_spec,
        pl.BlockSpec((D, din), lambda t, g: (0, 0)),
        pl.BlockSpec((H, D), lambda t, g: (0, 0)),
        pl.BlockSpec((H, D), lambda t, g: (0, 0)),
    ]
    args = [xin, wT, AsT, AdT]
    if has_b:
        in_specs.append(pl.BlockSpec((din,), lambda t, g: (0,)))
        args.append(b)
    return pl.pallas_call(
        body,
        grid=(T, B),
        in_specs=in_specs,
        out_specs=[
            pl.BlockSpec((1, 1, H, HC, NP), lambda t, g: (t, g, 0, 0, 0)),
            pl.BlockSpec((1, 1, H, NP), lambda t, g: (t, g, 0, 0)),
            pl.BlockSpec((1, 1, H, NP), lambda t, g: (t, g, 0, 0)),
        ],
        out_shape=[
            jax.ShapeDtypeStruct((T, B, H, HC, NP), jnp.float32),
            jax.ShapeDtypeStruct((T, B, H, NP), jnp.float32),
            jax.ShapeDtypeStruct((T, B, H, NP), jnp.float32),
        ],
    )(*args)


def _pool_gru_fc(out3, b3, wih, whh, bih, bhh, f1w, f1b, f2w, f2b):
    BA = B * A

    def body(o_ref, b3_ref, wih_ref, whh_ref, bih_ref, bhh_ref, f1w_ref,
             f1b_ref, f2w_ref, f2b_ref, pred_ref, h_ref):
        t = pl.program_id(0)

        @pl.when(t == 0)
        def _():
            h_ref[...] = jnp.zeros_like(h_ref)

        b3c = b3_ref[...].reshape(D, 1)
        ags = []
        gms = []
        for g in range(B):
            xg = jnp.maximum(o_ref[0, g].reshape(D, NP) + b3c, 0.0)
            ags.append(xg[:, :A])
            gm = jnp.sum(xg[:, :NPG], axis=1, keepdims=True) * (1.0 / NPG)
            gms.append(jnp.broadcast_to(gm, (D, A)))
        combT = jnp.concatenate(
            [jnp.concatenate(ags, axis=1), jnp.concatenate(gms, axis=1)],
            axis=0)
        hprev = h_ref[...]
        gi = jnp.dot(wih_ref[...], combT,
                     preferred_element_type=jnp.float32) + bih_ref[...].reshape(3 * HID, 1)
        gh = jnp.dot(whh_ref[...], hprev,
                     preferred_element_type=jnp.float32) + bhh_ref[...].reshape(3 * HID, 1)
        ir, iz, inn = gi[:HID], gi[HID:2 * HID], gi[2 * HID:]
        hr, hz, hn = gh[:HID], gh[HID:2 * HID], gh[2 * HID:]
        rg = jax.nn.sigmoid(ir + hr)
        zg = jax.nn.sigmoid(iz + hz)
        ng = jnp.tanh(inn + rg * hn)
        hnew = (1.0 - zg) * ng + zg * hprev
        h_ref[...] = hnew
        f1 = jnp.maximum(
            jnp.dot(f1w_ref[...], hnew,
                    preferred_element_type=jnp.float32) + f1b_ref[...].reshape(HC, 1), 0.0)
        pred = jnp.dot(f2w_ref[...], f1,
                       preferred_element_type=jnp.float32) + f2b_ref[...].reshape(OUT, 1)
        pred_ref[0] = pred

    full = lambda shape: pl.BlockSpec(shape, lambda t: tuple(0 for _ in shape))
    return pl.pallas_call(
        body,
        grid=(T,),
        in_specs=[
            pl.BlockSpec((1, B, H, HC, NP), lambda t: (t, 0, 0, 0, 0)),
            full((D,)),
            full((3 * HID, 2 * D)),
            full((3 * HID, HID)),
            full((3 * HID,)),
            full((3 * HID,)),
            full((HC, HID)),
            full((HC,)),
            full((OUT, HC)),
            full((OUT,)),
        ],
        out_specs=pl.BlockSpec((1, OUT, BA), lambda t: (t, 0, 0)),
        out_shape=jax.ShapeDtypeStruct((T, OUT, BA), jnp.float32),
        scratch_shapes=[pltpu.VMEM((HID, BA), jnp.float32)],
    )(out3, b3, wih, whh, bih, bhh, f1w, f1b, f2w, f2b)


def _block_diag_projT(a):
    return (jnp.eye(H, dtype=a.dtype)[:, :, None] * a[:, None, :]).reshape(H, D)


_loopattr_k = _make_loopattr()
_gat_nl = _make_gat(False)
_gat_sl = _make_gat(True)


def kernel(x, edge_index, edge_attr, batch, w1, as1, ad1, ae1, we1, b1, w2,
           as2, ad2, ae2, we2, b2, w3, as3, ad3, ae3, we3, b3, wih, whh, bih,
           bhh, fc1w, fc1b, fc2w, fc2b):
    offs = (jnp.arange(B, dtype=jnp.int32) * NPG)[None, :, None]
    src_h = edge_index[:, 0].reshape(T, B, EPG) - offs
    dst_h = edge_index[:, 1].reshape(T, B, EPG) - offs
    ea_h = edge_attr.reshape(T, B, EPG)

    kaps = []
    for we_l, ae_l in ((we1, ae1), (we2, ae2), (we3, ae3)):
        kap = (we_l.reshape(H, HC) * ae_l).sum(-1)
        kaps.append(jnp.broadcast_to(kap[:, None], (H, HC)))

    la_h = _loopattr_k(dst_h, ea_h)

    xT = x.reshape(T, B, NPG, IN).transpose(0, 1, 3, 2)
    xT = jnp.pad(xT, ((0, 0), (0, 0), (0, 0), (0, NP - NPG)))

    xin = xT
    layers = (
        (w1, as1, ad1, None, kaps[0], False),
        (w2, as2, ad2, b1, kaps[1], True),
        (w3, as3, ad3, b2, kaps[2], True),
    )
    for w_l, as_l, ad_l, b_prev, kap_l, sl in layers:
        xs, asrc, adst = _dense_tc(xin, w_l.T, _block_diag_projT(as_l),
                                   _block_diag_projT(ad_l), b_prev)
        if sl:
            xin = _gat_sl(src_h, dst_h, ea_h, asrc, adst, xs, kap_l, la_h)
        else:
            xin = _gat_nl(src_h, dst_h, ea_h, asrc, adst, xs, kap_l)

    preds = _pool_gru_fc(xin, b3, wih, whh, bih, bhh, fc1w, fc1b, fc2w, fc2b)
    return preds.reshape(T, OUT, B, A).transpose(2, 0, 3, 1)
